# baseline (speedup 1.0000x reference)
import functools

import jax
import jax.numpy as jnp
from jax import lax
from jax.experimental import pallas as pl
from jax.experimental.pallas import tpu as pltpu
from jax.experimental.pallas import tpu_sc as plsc

N = 10000
P = 640000
M = 2000
L = 50000

NC = 2
NS = 16
NT = NC * NS
PT = P // NT
G = 100
NCH = PT // G
NPAD = 10112
MPAD = 2048
CT = 1568
LPAD = CT * NT

_MESH = plsc.VectorSubcoreMesh(
    core_axis_name="c", subcore_axis_name="s", num_cores=NC, num_subcores=NS)
_SC_PARAMS = pltpu.CompilerParams(needs_layout_passes=False,
                                  use_tc_tiling_on_sc=False)


def _chunks(total, step):
  off = 0
  while off < total:
    sz = min(step, total - off)
    yield off, sz
    off += sz


@functools.partial(
    pl.kernel,
    out_type=(jax.ShapeDtypeStruct((NT, 1, N), jnp.float32),
              jax.ShapeDtypeStruct((NT, 1, M), jnp.float32)),
    mesh=_MESH,
    compiler_params=_SC_PARAMS,
    scratch_types=[
        pltpu.VMEM((PT,), jnp.int32),
        pltpu.VMEM((PT,), jnp.int32),
        pltpu.VMEM((N,), jnp.float32),
        pltpu.VMEM((M,), jnp.float32),
    ],
)
def _degrees(pn_hbm, pe_hbm, dv_out, de_out, pn_v, pe_v, dv_v, de_v):
  c = lax.axis_index("c")
  s = lax.axis_index("s")
  wid = s * NC + c
  base = wid * PT

  zero = jnp.zeros((16,), jnp.float32)

  def _zero_dv(i, carry):
    dv_v[pl.ds(i * 16, 16)] = zero
    return carry

  def _zero_de(i, carry):
    de_v[pl.ds(i * 16, 16)] = zero
    return carry

  lax.fori_loop(0, N // 16, _zero_dv, 0)
  lax.fori_loop(0, M // 16, _zero_de, 0)

  pltpu.sync_copy(pn_hbm.at[pl.ds(base, PT)], pn_v)
  pltpu.sync_copy(pe_hbm.at[pl.ds(base, PT)], pe_v)

  ones = jnp.ones((16,), jnp.float32)

  def _count(i, carry):
    idx_n = pn_v[pl.ds(i * 16, 16)]
    plsc.addupdate_scatter(dv_v, [idx_n], ones)
    idx_e = pe_v[pl.ds(i * 16, 16)]
    plsc.addupdate_scatter(de_v, [idx_e], ones)
    return carry

  lax.fori_loop(0, PT // 16, _count, 0)

  pltpu.sync_copy(dv_v, dv_out.at[wid].at[0])
  pltpu.sync_copy(de_v, de_out.at[wid].at[0])


def _make_seg(n_dst, d):
  rps = (n_dst // NS + 7) // 8 * 8
  n_pad = rps * NS
  stage = n_dst == N
  nb = 5

  @functools.partial(
      pl.kernel,
      out_type=jax.ShapeDtypeStruct((NC, n_pad, d), jnp.float32),
      mesh=_MESH,
      compiler_params=_SC_PARAMS,
      scratch_types=[
          pltpu.VMEM((NCH, G), jnp.int32),
          pltpu.VMEM((NCH, G), jnp.int32),
          [pltpu.VMEM((G, d), jnp.float32)] * nb,
          [pltpu.SemaphoreType.DMA] * nb,
          [pltpu.SemaphoreType.DMA] * nb,
          pltpu.VMEM_SHARED((n_pad, d), jnp.float32),
      ] + ([pltpu.VMEM_SHARED((MPAD, d), jnp.float32)] if stage else []),
      name=f"seg_{n_dst}_{d}",
  )
  def seg(src_hbm, gidx_hbm, sidx_hbm, out_hbm, gidx_v, sidx_v, rows,
          sem_g, sem_s, acc, *maybe_table):
    table = maybe_table[0] if stage else src_hbm
    c = lax.axis_index("c")
    s = lax.axis_index("s")
    wid = s * NC + c
    row0 = s * rps

    zero = jnp.zeros((16,), jnp.float32)

    def _zrow(r, carry):
      def _zcol(j, carry2):
        rows[0][r, pl.ds(j * 16, 16)] = zero
        return carry2
      return lax.fori_loop(0, d // 16, _zcol, carry)

    lax.fori_loop(0, G, _zrow, 0)

    for off, sz in _chunks(rps, G):
      pltpu.sync_copy(rows[0].at[pl.ds(0, sz)], acc.at[pl.ds(row0 + off, sz)])
    if stage:
      spt = MPAD // NS
      pltpu.sync_copy(src_hbm.at[pl.ds(s * spt, spt)],
                      table.at[pl.ds(s * spt, spt)])
    plsc.subcore_barrier()

    pltpu.sync_copy(gidx_hbm.at[wid], gidx_v)
    pltpu.sync_copy(sidx_hbm.at[wid], sidx_v)

    for b in range(nb):
      pltpu.async_copy(table.at[gidx_v.at[b]], rows[b], sem_g[b])

    def _super(i, carry):
      for b in range(nb):
        t = i * nb + b
        pltpu.make_async_copy(table.at[gidx_v.at[t]], rows[b],
                              sem_g[b]).wait()
        pltpu.async_copy(rows[b], acc.at[sidx_v.at[t]], sem_s[b], add=True)
        pltpu.make_async_copy(rows[b], acc.at[sidx_v.at[t]], sem_s[b]).wait()
        pltpu.async_copy(table.at[gidx_v.at[t + nb]], rows[b], sem_g[b])
      return carry

    lax.fori_loop(0, NCH // nb - 1, _super, 0)

    last = NCH - nb
    for b in range(nb):
      t = last + b
      pltpu.make_async_copy(table.at[gidx_v.at[t]], rows[b],
                            sem_g[b]).wait()
      pltpu.async_copy(rows[b], acc.at[sidx_v.at[t]], sem_s[b], add=True)
    for b in range(nb):
      t = last + b
      pltpu.make_async_copy(rows[b], acc.at[sidx_v.at[t]], sem_s[b]).wait()
    plsc.subcore_barrier()

    pltpu.sync_copy(acc.at[pl.ds(row0, rps)],
                    out_hbm.at[c].at[pl.ds(row0, rps)])

  return seg


_seg_e128 = _make_seg(M, 128)
_seg_v64 = _make_seg(N, 64)
_seg_e64 = _make_seg(M, 64)


@functools.partial(
    pl.kernel,
    out_type=jax.ShapeDtypeStruct((LPAD,), jnp.float32),
    mesh=_MESH,
    compiler_params=_SC_PARAMS,
    scratch_types=[
        pltpu.VMEM((N,), jnp.float32),
        pltpu.VMEM((CT,), jnp.int32),
        pltpu.VMEM((CT,), jnp.int32),
        pltpu.VMEM((CT,), jnp.float32),
    ],
)
def _link(s_hbm, u_hbm, v_hbm, out_hbm, s_v, u_v, v_v, o_v):
  c = lax.axis_index("c")
  s = lax.axis_index("s")
  wid = s * NC + c
  base = wid * CT

  pltpu.sync_copy(s_hbm, s_v)
  pltpu.sync_copy(u_hbm.at[pl.ds(base, CT)], u_v)
  pltpu.sync_copy(v_hbm.at[pl.ds(base, CT)], v_v)

  def _it(i, carry):
    u16 = u_v[pl.ds(i * 16, 16)]
    v16 = v_v[pl.ds(i * 16, 16)]
    su = plsc.load_gather(s_v, [u16])
    sv = plsc.load_gather(s_v, [v16])
    x = 0.5 * (su + sv)
    o_v[pl.ds(i * 16, 16)] = 1.0 / (1.0 + jnp.exp(-x))
    return carry

  lax.fori_loop(0, CT // 16, _it, 0)
  pltpu.sync_copy(o_v, out_hbm.at[pl.ds(base, CT)])


def _inv_sqrt_dv(dvp):
  d = jnp.sum(dvp, axis=0)
  return jnp.where(d > 0, 1.0 / jnp.sqrt(jnp.maximum(d, 1e-12)), 0.0)


def _tc_scale_in(x_ref, w_ref, b_ref, dvp_ref, o_ref):
  isd = _inv_sqrt_dv(dvp_ref[...])
  y = jnp.dot(x_ref[...], w_ref[...], preferred_element_type=jnp.float32)
  o_ref[:N, :] = (y + b_ref[...][None, :]) * isd[:, None]
  o_ref[N:, :] = jnp.zeros((NPAD - N, 128), jnp.float32)


def _tc_scale_e2(xep_ref, dep_ref, lo_ref, hi_ref):
  d = jnp.sum(dep_ref[...], axis=0)
  inv = jnp.where(d > 0, 1.0 / jnp.maximum(d, 1e-12), 0.0)
  y = (xep_ref[0, :M, :] + xep_ref[1, :M, :]) * inv[:, None]
  zpad = jnp.zeros((MPAD - M, 64), jnp.float32)
  lo_ref[:M, :] = y[:, :64]
  lo_ref[M:, :] = zpad
  hi_ref[:M, :] = y[:, 64:]
  hi_ref[M:, :] = zpad


def _tc_scale_e(xep_ref, dep_ref, o_ref):
  d = jnp.sum(dep_ref[...], axis=0)
  inv = jnp.where(d > 0, 1.0 / jnp.maximum(d, 1e-12), 0.0)
  o_ref[:M, :] = (xep_ref[0, :M, :] + xep_ref[1, :M, :]) * inv[:, None]
  o_ref[M:, :] = jnp.zeros((MPAD - M, o_ref.shape[1]), jnp.float32)


def _tc_mid(lo_ref, hi_ref, dvp_ref, w_ref, b_ref, o_ref):
  isd = _inv_sqrt_dv(dvp_ref[...])
  h_lo = jax.nn.relu((lo_ref[0, :N, :] + lo_ref[1, :N, :]) * isd[:, None])
  h_hi = jax.nn.relu((hi_ref[0, :N, :] + hi_ref[1, :N, :]) * isd[:, None])
  y = (jnp.dot(h_lo, w_ref[:64, :], preferred_element_type=jnp.float32)
       + jnp.dot(h_hi, w_ref[64:, :], preferred_element_type=jnp.float32))
  o_ref[:N, :] = (y + b_ref[...][None, :]) * isd[:, None]
  o_ref[N:, :] = jnp.zeros((NPAD - N, o_ref.shape[1]), jnp.float32)


def _tc_final(xvp_ref, dvp_ref, wfc_ref, bfc_ref, o_ref):
  isd = _inv_sqrt_dv(dvp_ref[...])
  h = jax.nn.relu((xvp_ref[0, :N, :] + xvp_ref[1, :N, :]) * isd[:, None])
  o_ref[...] = jnp.sum(h * wfc_ref[...][:, 0][None, :], axis=1) + bfc_ref[0]


def _tc(body, out_shape, *args):
  return pl.pallas_call(body, out_shape=out_shape)(*args)


def kernel(X, pair_nodes, pair_edges, link_edges, W1, b1, W2, b2, Wfc, bfc):
  pn = pair_nodes.astype(jnp.int32)
  pe = pair_edges.astype(jnp.int32)
  le = link_edges.astype(jnp.int32)
  pn2 = pn.reshape(NT, NCH, G)
  pe2 = pe.reshape(NT, NCH, G)

  dvp3, dep3 = _degrees(pn, pe)
  dvp = dvp3.reshape(NT, N)
  dep = dep3.reshape(NT, M)

  xn = _tc(_tc_scale_in, jax.ShapeDtypeStruct((NPAD, 128), jnp.float32),
           X, W1, b1, dvp)
  xep = _seg_e128(xn, pn2, pe2)
  ehalf = jax.ShapeDtypeStruct((MPAD, 64), jnp.float32)
  xe_lo, xe_hi = _tc(_tc_scale_e2, (ehalf, ehalf), xep, dep)
  xvp_lo = _seg_v64(xe_lo, pe2, pn2)
  xvp_hi = _seg_v64(xe_hi, pe2, pn2)
  xn2 = _tc(_tc_mid, jax.ShapeDtypeStruct((NPAD, 64), jnp.float32),
            xvp_lo, xvp_hi, dvp, W2, b2)
  xep2 = _seg_e64(xn2, pn2, pe2)
  xe2 = _tc(_tc_scale_e, jax.ShapeDtypeStruct((MPAD, 64), jnp.float32),
            xep2, dep)
  xvp2 = _seg_v64(xe2, pe2, pn2)
  s = _tc(_tc_final, jax.ShapeDtypeStruct((N,), jnp.float32),
          xvp2, dvp, Wfc, bfc)

  u = jnp.pad(le[:, 0], (0, LPAD - L))
  v = jnp.pad(le[:, 1], (0, LPAD - L))
  out = _link(s, u, v)
  return out[:L].reshape(L, 1)


# speedup vs baseline: 1.1019x; 1.1019x over previous
import functools

import jax
import jax.numpy as jnp
from jax import lax
from jax.experimental import pallas as pl
from jax.experimental.pallas import tpu as pltpu
from jax.experimental.pallas import tpu_sc as plsc

N = 10000
P = 640000
M = 2000
L = 50000

NC = 2
NS = 16
NT = NC * NS
PT = P // NT
G = 100
NCH = PT // G
NPAD = 10112
MPAD = 2048
CT = 1568
LPAD = CT * NT

_MESH = plsc.VectorSubcoreMesh(
    core_axis_name="c", subcore_axis_name="s", num_cores=NC, num_subcores=NS)
_SC_PARAMS = pltpu.CompilerParams(needs_layout_passes=False,
                                  use_tc_tiling_on_sc=False)


def _chunks(total, step):
  off = 0
  while off < total:
    sz = min(step, total - off)
    yield off, sz
    off += sz


@functools.partial(
    pl.kernel,
    out_type=(jax.ShapeDtypeStruct((NT, 1, N), jnp.float32),
              jax.ShapeDtypeStruct((NT, 1, M), jnp.float32)),
    mesh=_MESH,
    compiler_params=_SC_PARAMS,
    scratch_types=[
        pltpu.VMEM((PT,), jnp.int32),
        pltpu.VMEM((PT,), jnp.int32),
        pltpu.VMEM((N,), jnp.float32),
        pltpu.VMEM((M,), jnp.float32),
    ],
)
def _degrees(pn_hbm, pe_hbm, dv_out, de_out, pn_v, pe_v, dv_v, de_v):
  c = lax.axis_index("c")
  s = lax.axis_index("s")
  wid = s * NC + c
  base = wid * PT

  zero = jnp.zeros((16,), jnp.float32)

  def _zero_dv(i, carry):
    dv_v[pl.ds(i * 16, 16)] = zero
    return carry

  def _zero_de(i, carry):
    de_v[pl.ds(i * 16, 16)] = zero
    return carry

  lax.fori_loop(0, N // 16, _zero_dv, 0)
  lax.fori_loop(0, M // 16, _zero_de, 0)

  pltpu.sync_copy(pn_hbm.at[pl.ds(base, PT)], pn_v)
  pltpu.sync_copy(pe_hbm.at[pl.ds(base, PT)], pe_v)

  ones = jnp.ones((16,), jnp.float32)

  def _count(i, carry):
    idx_n = pn_v[pl.ds(i * 16, 16)]
    plsc.addupdate_scatter(dv_v, [idx_n], ones)
    idx_e = pe_v[pl.ds(i * 16, 16)]
    plsc.addupdate_scatter(de_v, [idx_e], ones)
    return carry

  lax.fori_loop(0, PT // 16, _count, 0)

  pltpu.sync_copy(dv_v, dv_out.at[wid].at[0])
  pltpu.sync_copy(de_v, de_out.at[wid].at[0])


def _make_seg(n_dst, d):
  rps = (n_dst // NS + 7) // 8 * 8
  n_pad = rps * NS
  nb = 5

  @functools.partial(
      pl.kernel,
      out_type=jax.ShapeDtypeStruct((NC, n_pad, d), jnp.float32),
      mesh=_MESH,
      compiler_params=_SC_PARAMS,
      scratch_types=[
          pltpu.VMEM((NCH, G), jnp.int32),
          pltpu.VMEM((NCH, G), jnp.int32),
          [pltpu.VMEM((G, d), jnp.float32)] * nb,
          [pltpu.SemaphoreType.DMA] * nb,
          [pltpu.SemaphoreType.DMA] * nb,
          pltpu.VMEM_SHARED((n_pad, d), jnp.float32),
      ],
      name=f"seg_{n_dst}_{d}",
  )
  def seg(src_hbm, gidx_hbm, sidx_hbm, out_hbm, gidx_v, sidx_v, rows,
          sem_g, sem_s, acc):
    table = src_hbm
    c = lax.axis_index("c")
    s = lax.axis_index("s")
    wid = s * NC + c
    row0 = s * rps

    zero = jnp.zeros((16,), jnp.float32)

    def _zrow(r, carry):
      def _zcol(j, carry2):
        rows[0][r, pl.ds(j * 16, 16)] = zero
        return carry2
      return lax.fori_loop(0, d // 16, _zcol, carry)

    lax.fori_loop(0, G, _zrow, 0)

    for off, sz in _chunks(rps, G):
      pltpu.sync_copy(rows[0].at[pl.ds(0, sz)], acc.at[pl.ds(row0 + off, sz)])
    plsc.subcore_barrier()

    pltpu.sync_copy(gidx_hbm.at[wid], gidx_v)
    pltpu.sync_copy(sidx_hbm.at[wid], sidx_v)

    for b in range(nb):
      pltpu.async_copy(table.at[gidx_v.at[b]], rows[b], sem_g[b])

    def _super(i, carry):
      for b in range(nb):
        t = i * nb + b
        pltpu.make_async_copy(table.at[gidx_v.at[t]], rows[b],
                              sem_g[b]).wait()
        pltpu.async_copy(rows[b], acc.at[sidx_v.at[t]], sem_s[b], add=True)
        pltpu.make_async_copy(rows[b], acc.at[sidx_v.at[t]], sem_s[b]).wait()
        pltpu.async_copy(table.at[gidx_v.at[t + nb]], rows[b], sem_g[b])
      return carry

    lax.fori_loop(0, NCH // nb - 1, _super, 0)

    last = NCH - nb
    for b in range(nb):
      t = last + b
      pltpu.make_async_copy(table.at[gidx_v.at[t]], rows[b],
                            sem_g[b]).wait()
      pltpu.async_copy(rows[b], acc.at[sidx_v.at[t]], sem_s[b], add=True)
    for b in range(nb):
      t = last + b
      pltpu.make_async_copy(rows[b], acc.at[sidx_v.at[t]], sem_s[b]).wait()
    plsc.subcore_barrier()

    pltpu.sync_copy(acc.at[pl.ds(row0, rps)],
                    out_hbm.at[c].at[pl.ds(row0, rps)])

  return seg


_seg_e128 = _make_seg(M, 128)
_seg_v64 = _make_seg(N, 64)
_seg_e64 = _make_seg(M, 64)


@functools.partial(
    pl.kernel,
    out_type=jax.ShapeDtypeStruct((LPAD,), jnp.float32),
    mesh=_MESH,
    compiler_params=_SC_PARAMS,
    scratch_types=[
        pltpu.VMEM((N,), jnp.float32),
        pltpu.VMEM((CT,), jnp.int32),
        pltpu.VMEM((CT,), jnp.int32),
        pltpu.VMEM((CT,), jnp.float32),
    ],
)
def _link(s_hbm, u_hbm, v_hbm, out_hbm, s_v, u_v, v_v, o_v):
  c = lax.axis_index("c")
  s = lax.axis_index("s")
  wid = s * NC + c
  base = wid * CT

  pltpu.sync_copy(s_hbm, s_v)
  pltpu.sync_copy(u_hbm.at[pl.ds(base, CT)], u_v)
  pltpu.sync_copy(v_hbm.at[pl.ds(base, CT)], v_v)

  def _it(i, carry):
    u16 = u_v[pl.ds(i * 16, 16)]
    v16 = v_v[pl.ds(i * 16, 16)]
    su = plsc.load_gather(s_v, [u16])
    sv = plsc.load_gather(s_v, [v16])
    x = 0.5 * (su + sv)
    o_v[pl.ds(i * 16, 16)] = 1.0 / (1.0 + jnp.exp(-x))
    return carry

  lax.fori_loop(0, CT // 16, _it, 0)
  pltpu.sync_copy(o_v, out_hbm.at[pl.ds(base, CT)])


def _inv_sqrt_dv(dvp):
  d = jnp.sum(dvp, axis=0)
  return jnp.where(d > 0, 1.0 / jnp.sqrt(jnp.maximum(d, 1e-12)), 0.0)


def _tc_scale_in(x_ref, w_ref, b_ref, dvp_ref, o_ref):
  isd = _inv_sqrt_dv(dvp_ref[...])
  y = jnp.dot(x_ref[...], w_ref[...], preferred_element_type=jnp.float32)
  o_ref[:N, :] = (y + b_ref[...][None, :]) * isd[:, None]
  o_ref[N:, :] = jnp.zeros((NPAD - N, 128), jnp.float32)


def _tc_scale_e2(xep_ref, dep_ref, lo_ref, hi_ref):
  d = jnp.sum(dep_ref[...], axis=0)
  inv = jnp.where(d > 0, 1.0 / jnp.maximum(d, 1e-12), 0.0)
  y = (xep_ref[0, :M, :] + xep_ref[1, :M, :]) * inv[:, None]
  zpad = jnp.zeros((MPAD - M, 64), jnp.float32)
  lo_ref[:M, :] = y[:, :64]
  lo_ref[M:, :] = zpad
  hi_ref[:M, :] = y[:, 64:]
  hi_ref[M:, :] = zpad


def _tc_scale_e(xep_ref, dep_ref, o_ref):
  d = jnp.sum(dep_ref[...], axis=0)
  inv = jnp.where(d > 0, 1.0 / jnp.maximum(d, 1e-12), 0.0)
  o_ref[:M, :] = (xep_ref[0, :M, :] + xep_ref[1, :M, :]) * inv[:, None]
  o_ref[M:, :] = jnp.zeros((MPAD - M, o_ref.shape[1]), jnp.float32)


def _tc_mid(lo_ref, hi_ref, dvp_ref, w_ref, b_ref, o_ref):
  isd = _inv_sqrt_dv(dvp_ref[...])
  h_lo = jax.nn.relu((lo_ref[0, :N, :] + lo_ref[1, :N, :]) * isd[:, None])
  h_hi = jax.nn.relu((hi_ref[0, :N, :] + hi_ref[1, :N, :]) * isd[:, None])
  y = (jnp.dot(h_lo, w_ref[:64, :], preferred_element_type=jnp.float32)
       + jnp.dot(h_hi, w_ref[64:, :], preferred_element_type=jnp.float32))
  o_ref[:N, :] = (y + b_ref[...][None, :]) * isd[:, None]
  o_ref[N:, :] = jnp.zeros((NPAD - N, o_ref.shape[1]), jnp.float32)


def _tc_final(xvp_ref, dvp_ref, wfc_ref, bfc_ref, o_ref):
  isd = _inv_sqrt_dv(dvp_ref[...])
  h = jax.nn.relu((xvp_ref[0, :N, :] + xvp_ref[1, :N, :]) * isd[:, None])
  o_ref[...] = jnp.sum(h * wfc_ref[...][:, 0][None, :], axis=1) + bfc_ref[0]


def _tc(body, out_shape, *args):
  return pl.pallas_call(body, out_shape=out_shape)(*args)


def kernel(X, pair_nodes, pair_edges, link_edges, W1, b1, W2, b2, Wfc, bfc):
  pn = pair_nodes.astype(jnp.int32)
  pe = pair_edges.astype(jnp.int32)
  le = link_edges.astype(jnp.int32)
  pn2 = pn.reshape(NT, NCH, G)
  pe2 = pe.reshape(NT, NCH, G)

  dvp3, dep3 = _degrees(pn, pe)
  dvp = dvp3.reshape(NT, N)
  dep = dep3.reshape(NT, M)

  xn = _tc(_tc_scale_in, jax.ShapeDtypeStruct((NPAD, 128), jnp.float32),
           X, W1, b1, dvp)
  xep = _seg_e128(xn, pn2, pe2)
  ehalf = jax.ShapeDtypeStruct((MPAD, 64), jnp.float32)
  xe_lo, xe_hi = _tc(_tc_scale_e2, (ehalf, ehalf), xep, dep)
  xvp_lo = _seg_v64(xe_lo, pe2, pn2)
  xvp_hi = _seg_v64(xe_hi, pe2, pn2)
  xn2 = _tc(_tc_mid, jax.ShapeDtypeStruct((NPAD, 64), jnp.float32),
            xvp_lo, xvp_hi, dvp, W2, b2)
  xep2 = _seg_e64(xn2, pn2, pe2)
  xe2 = _tc(_tc_scale_e, jax.ShapeDtypeStruct((MPAD, 64), jnp.float32),
            xep2, dep)
  xvp2 = _seg_v64(xe2, pe2, pn2)
  s = _tc(_tc_final, jax.ShapeDtypeStruct((N,), jnp.float32),
          xvp2, dvp, Wfc, bfc)

  u = jnp.pad(le[:, 0], (0, LPAD - L))
  v = jnp.pad(le[:, 1], (0, LPAD - L))
  out = _link(s, u, v)
  return out[:L].reshape(L, 1)
